# baseline (speedup 1.0000x reference)
import jax
import jax.numpy as jnp
from jax import lax
from jax.experimental import pallas as pl
from jax.experimental.pallas import tpu as pltpu
from jax.experimental.pallas import tpu_sc as plsc

N = 10000
D = 128
E = 320000
H = 128

NC = 2
NS = 16
NW = NC * NS
EPW = E // NW
CH = 64
NCHUNK = EPW // CH
TAIL = EPW - NCHUNK * CH
NP = 10048
NT = NP // CH

RB = 1000
LANES = 16


def _pack_bf16(o):
    r = o.shape[0]
    g = o.reshape(r, H // 32, 2, 16)
    lo = jax.lax.bitcast_convert_type(
        g[:, :, 0, :].astype(jnp.bfloat16), jnp.uint16).astype(jnp.uint32)
    hi = jax.lax.bitcast_convert_type(
        g[:, :, 1, :].astype(jnp.bfloat16), jnp.uint16).astype(jnp.uint32)
    word = lo | (hi << 16)
    return jax.lax.bitcast_convert_type(word, jnp.int32).reshape(r, H // 2)


def _proj_body(x_ref, h_ref, wx_ref, wh_ref, t_ref, s_ref):
    o = (jnp.dot(x_ref[...], wx_ref[...], preferred_element_type=jnp.float32)
         + jnp.dot(h_ref[...], wh_ref[...], preferred_element_type=jnp.float32))
    t_ref[...] = jnp.concatenate(
        [_pack_bf16(o[:, :H]), _pack_bf16(o[:, H:2 * H])], axis=1)
    s_ref[...] = o[:, 2 * H:]


def _proj(x, hidden, wx, wh):
    grid = N // RB
    return pl.pallas_call(
        _proj_body,
        grid=(grid,),
        in_specs=[
            pl.BlockSpec((RB, D), lambda i: (i, 0)),
            pl.BlockSpec((RB, H), lambda i: (i, 0)),
            pl.BlockSpec((D, 3 * H), lambda i: (0, 0)),
            pl.BlockSpec((H, 3 * H), lambda i: (0, 0)),
        ],
        out_specs=[
            pl.BlockSpec((RB, H), lambda i: (i, 0)),
            pl.BlockSpec((RB, H), lambda i: (i, 0)),
        ],
        out_shape=[jax.ShapeDtypeStruct((N, H), jnp.int32),
                   jax.ShapeDtypeStruct((N, H), jnp.float32)],
    )(x, hidden, wx, wh)


def _edge_body(t_hbm, src_hbm, dst_hbm, out_hbm,
               a0, a1, b0, b1, m0, m1,
               is0, is1, id0, id1, d20, d21, it16, dt16, agg_sh,
               gsem0, gsem1, isem0, isem1, ssem0, ssem1):
    c = lax.axis_index("c")
    s = lax.axis_index("s")
    w = c * NS + s
    ebase = w * EPW

    a = (a0, a1)
    b = (b0, b1)
    m = (m0, m1)
    isv = (is0, is1)
    idv = (id0, id1)
    d2 = (d20, d21)
    gsem = (gsem0, gsem1)
    isem = (isem0, isem1)
    ssem = (ssem0, ssem1)

    def _zrow(r, carry):
        for k in range(H // LANES):
            m0[r, pl.ds(k * LANES, LANES)] = jnp.zeros((LANES,), jnp.float32)
        return carry
    lax.fori_loop(0, CH, _zrow, 0)

    ntiles = (NT - s + NS - 1) // NS

    def _ztile(t, carry):
        pltpu.sync_copy(m0, agg_sh.at[pl.ds((s + t * NS) * CH, CH)])
        return carry
    lax.fori_loop(0, ntiles, _ztile, 0)
    plsc.subcore_barrier()

    pltpu.sync_copy(src_hbm.at[pl.ds(ebase + NCHUNK * CH, TAIL)], it16)
    pltpu.sync_copy(dst_hbm.at[pl.ds(ebase + NCHUNK * CH, TAIL)], dt16)
    pltpu.async_copy(t_hbm.at[it16], a0.at[pl.ds(0, TAIL)], gsem0).wait()
    pltpu.async_copy(t_hbm.at[dt16], b0.at[pl.ds(0, TAIL)], gsem0).wait()

    mask_hi = jnp.full((LANES,), -65536, jnp.int32)

    def _relu_row(ar, br, mr, r):
        for j in range(H // 32):
            wa = ar[r, pl.ds(j * LANES, LANES)]
            wb = br[r, pl.ds(H // 2 + j * LANES, LANES)]
            alo = jax.lax.bitcast_convert_type(jnp.left_shift(wa, 16), jnp.float32)
            blo = jax.lax.bitcast_convert_type(jnp.left_shift(wb, 16), jnp.float32)
            ahi = jax.lax.bitcast_convert_type(wa & mask_hi, jnp.float32)
            bhi = jax.lax.bitcast_convert_type(wb & mask_hi, jnp.float32)
            mr[r, pl.ds(j * 32, LANES)] = jnp.maximum(alo + blo, 0.0)
            mr[r, pl.ds(j * 32 + LANES, LANES)] = jnp.maximum(ahi + bhi, 0.0)

    def _trow(r, carry):
        _relu_row(a0, b0, m0, r)
        return carry
    lax.fori_loop(0, TAIL, _trow, 0)
    pltpu.sync_copy(m0.at[pl.ds(0, TAIL)], agg_sh.at[dt16], add=True)

    def issue_idx(j, p):
        off = ebase + j * CH
        pltpu.async_copy(src_hbm.at[pl.ds(off, CH)], isv[p], isem[p])
        pltpu.async_copy(dst_hbm.at[pl.ds(off, CH)], idv[p], isem[p])

    def wait_idx(p):
        pltpu.make_async_copy(src_hbm.at[pl.ds(0, CH)], isv[p], isem[p]).wait()
        pltpu.make_async_copy(dst_hbm.at[pl.ds(0, CH)], idv[p], isem[p]).wait()

    def issue_gather(p):
        pltpu.async_copy(t_hbm.at[isv[p]], a[p], gsem[p])
        pltpu.async_copy(t_hbm.at[idv[p]], b[p], gsem[p])

    def wait_gather(p):
        pltpu.make_async_copy(t_hbm.at[isv[p]], a[p], gsem[p]).wait()
        pltpu.make_async_copy(t_hbm.at[idv[p]], b[p], gsem[p]).wait()

    def issue_scatter(p):
        pltpu.async_copy(m[p], agg_sh.at[d2[p]], ssem[p], add=True)

    def wait_scatter(p):
        pltpu.make_async_copy(m[p], agg_sh.at[d2[p]], ssem[p]).wait()

    def save_idx(p):
        for k in range(CH // LANES):
            sl = pl.ds(k * LANES, LANES)
            d2[p][sl] = idv[p][sl]

    def compute(p):
        ap, bp, mp = a[p], b[p], m[p]

        def _crow(r, carry):
            _relu_row(ap, bp, mp, r)
            return carry
        lax.fori_loop(0, CH, _crow, 0)

    issue_idx(0, 0)
    issue_idx(1, 1)
    wait_idx(0)
    issue_gather(0)

    def _macro(t, carry):
        @pl.when(t >= 1)
        def _():
            wait_scatter(0)
        wait_idx(1)
        issue_gather(1)
        wait_gather(0)
        save_idx(0)

        @pl.when(t < NCHUNK // 2 - 1)
        def _():
            issue_idx(2 * t + 2, 0)
        compute(0)
        issue_scatter(0)

        @pl.when(t >= 1)
        def _():
            wait_scatter(1)

        @pl.when(t < NCHUNK // 2 - 1)
        def _():
            wait_idx(0)
            issue_gather(0)
        wait_gather(1)
        save_idx(1)

        @pl.when(t < NCHUNK // 2 - 1)
        def _():
            issue_idx(2 * t + 3, 1)
        compute(1)
        issue_scatter(1)
        return carry
    lax.fori_loop(0, NCHUNK // 2, _macro, 0)

    wait_scatter(0)
    wait_scatter(1)
    plsc.subcore_barrier()

    def _wtile(t, carry):
        off = (s + t * NS) * CH
        pltpu.sync_copy(agg_sh.at[pl.ds(off, CH)], m0)
        pltpu.sync_copy(m0, out_hbm.at[c, pl.ds(off, CH)])
        return carry
    lax.fori_loop(0, ntiles, _wtile, 0)


def _edge(t, src, dst):
    mesh = plsc.VectorSubcoreMesh(core_axis_name="c", subcore_axis_name="s")
    f = pl.kernel(
        _edge_body,
        out_type=jax.ShapeDtypeStruct((NC, NP, H), jnp.float32),
        mesh=mesh,
        scratch_types=[
            pltpu.VMEM((CH, H), jnp.int32),
            pltpu.VMEM((CH, H), jnp.int32),
            pltpu.VMEM((CH, H), jnp.int32),
            pltpu.VMEM((CH, H), jnp.int32),
            pltpu.VMEM((CH, H), jnp.float32),
            pltpu.VMEM((CH, H), jnp.float32),
            pltpu.VMEM((CH,), jnp.int32),
            pltpu.VMEM((CH,), jnp.int32),
            pltpu.VMEM((CH,), jnp.int32),
            pltpu.VMEM((CH,), jnp.int32),
            pltpu.VMEM((CH,), jnp.int32),
            pltpu.VMEM((CH,), jnp.int32),
            pltpu.VMEM((TAIL,), jnp.int32),
            pltpu.VMEM((TAIL,), jnp.int32),
            pltpu.VMEM_SHARED((NP, H), jnp.float32),
            pltpu.SemaphoreType.DMA,
            pltpu.SemaphoreType.DMA,
            pltpu.SemaphoreType.DMA,
            pltpu.SemaphoreType.DMA,
            pltpu.SemaphoreType.DMA,
            pltpu.SemaphoreType.DMA,
        ],
    )
    return f(t, src, dst)


def _final_body(s_ref, agg_ref, w_ref, o_ref):
    a = agg_ref[0] + agg_ref[1]
    o_ref[...] = jnp.maximum(
        s_ref[...] + jnp.dot(a, w_ref[...], preferred_element_type=jnp.float32),
        0.0)


def _final(s, agg2, w_agg):
    grid = N // RB
    return pl.pallas_call(
        _final_body,
        grid=(grid,),
        in_specs=[
            pl.BlockSpec((RB, H), lambda i: (i, 0)),
            pl.BlockSpec((NC, RB, H), lambda i: (0, i, 0)),
            pl.BlockSpec((H, H), lambda i: (0, 0)),
        ],
        out_specs=pl.BlockSpec((RB, H), lambda i: (i, 0)),
        out_shape=jax.ShapeDtypeStruct((N, H), jnp.float32),
    )(s, agg2, w_agg)


def kernel(x, edge_index, hidden, W_src, W_dst, W_self, W_agg):
    ei = edge_index.astype(jnp.int32)
    src, dst = ei[0], ei[1]
    wx = jnp.concatenate([W_src[:D], W_dst[:D], W_self[:D]], axis=1)
    wh = jnp.concatenate([W_src[D:], W_dst[D:], W_self[D:]], axis=1)
    t, s = _proj(x, hidden, wx, wh)
    agg2 = _edge(t, src, dst)
    return _final(s, agg2, W_agg)


# speedup vs baseline: 1.4457x; 1.4457x over previous
import jax
import jax.numpy as jnp
from jax import lax
from jax.experimental import pallas as pl
from jax.experimental.pallas import tpu as pltpu
from jax.experimental.pallas import tpu_sc as plsc

N = 10000
D = 128
E = 320000
H = 128

NC = 2
NS = 16
NW = NC * NS
EPW = E // NW
CH = 64
NCHUNK = EPW // CH
TAIL = EPW - NCHUNK * CH
NP = 10048
NT = NP // CH

RB = 1000
LANES = 16


def _proj_body(x_ref, h_ref, wx_ref, wh_ref, ps_ref, pd_ref, s_ref):
    o = (jnp.dot(x_ref[...], wx_ref[...], preferred_element_type=jnp.float32)
         + jnp.dot(h_ref[...], wh_ref[...], preferred_element_type=jnp.float32))
    ps_ref[...] = o[:, :H]
    pd_ref[...] = o[:, H:2 * H]
    s_ref[...] = o[:, 2 * H:]


def _proj(x, hidden, wx, wh):
    grid = N // RB
    return pl.pallas_call(
        _proj_body,
        grid=(grid,),
        in_specs=[
            pl.BlockSpec((RB, D), lambda i: (i, 0)),
            pl.BlockSpec((RB, H), lambda i: (i, 0)),
            pl.BlockSpec((D, 3 * H), lambda i: (0, 0)),
            pl.BlockSpec((H, 3 * H), lambda i: (0, 0)),
        ],
        out_specs=[
            pl.BlockSpec((RB, H), lambda i: (i, 0)),
            pl.BlockSpec((RB, H), lambda i: (i, 0)),
            pl.BlockSpec((RB, H), lambda i: (i, 0)),
        ],
        out_shape=[jax.ShapeDtypeStruct((N, H), jnp.float32)] * 3,
    )(x, hidden, wx, wh)


def _edge_body(ps_hbm, pd_hbm, src_hbm, dst_hbm, out_hbm,
               a0, a1, b0, b1, m0, m1,
               is0, is1, id0, id1, d20, d21, it16, dt16, agg_sh,
               gsem0, gsem1, isem0, isem1, ssem0, ssem1):
    c = lax.axis_index("c")
    s = lax.axis_index("s")
    w = c * NS + s
    ebase = w * EPW

    a = (a0, a1)
    b = (b0, b1)
    m = (m0, m1)
    isv = (is0, is1)
    idv = (id0, id1)
    d2 = (d20, d21)
    gsem = (gsem0, gsem1)
    isem = (isem0, isem1)
    ssem = (ssem0, ssem1)

    def _zrow(r, carry):
        for k in range(H // LANES):
            m0[r, pl.ds(k * LANES, LANES)] = jnp.zeros((LANES,), jnp.float32)
        return carry
    lax.fori_loop(0, CH, _zrow, 0)

    ntiles = (NT - s + NS - 1) // NS

    def _ztile(t, carry):
        pltpu.sync_copy(m0, agg_sh.at[pl.ds((s + t * NS) * CH, CH)])
        return carry
    lax.fori_loop(0, ntiles, _ztile, 0)
    plsc.subcore_barrier()

    pltpu.sync_copy(src_hbm.at[pl.ds(ebase + NCHUNK * CH, TAIL)], it16)
    pltpu.sync_copy(dst_hbm.at[pl.ds(ebase + NCHUNK * CH, TAIL)], dt16)
    pltpu.async_copy(ps_hbm.at[it16], a0.at[pl.ds(0, TAIL)], gsem0).wait()
    pltpu.async_copy(pd_hbm.at[dt16], b0.at[pl.ds(0, TAIL)], gsem0).wait()

    def _trow(r, carry):
        for k in range(H // LANES):
            sl = pl.ds(k * LANES, LANES)
            m0[r, sl] = jnp.maximum(a0[r, sl] + b0[r, sl], 0.0)
        return carry
    lax.fori_loop(0, TAIL, _trow, 0)
    pltpu.sync_copy(m0.at[pl.ds(0, TAIL)], agg_sh.at[dt16], add=True)

    def issue_idx(j, p):
        off = ebase + j * CH
        pltpu.async_copy(src_hbm.at[pl.ds(off, CH)], isv[p], isem[p])
        pltpu.async_copy(dst_hbm.at[pl.ds(off, CH)], idv[p], isem[p])

    def wait_idx(p):
        pltpu.make_async_copy(src_hbm.at[pl.ds(0, CH)], isv[p], isem[p]).wait()
        pltpu.make_async_copy(dst_hbm.at[pl.ds(0, CH)], idv[p], isem[p]).wait()

    def issue_gather(p):
        pltpu.async_copy(ps_hbm.at[isv[p]], a[p], gsem[p])
        pltpu.async_copy(pd_hbm.at[idv[p]], b[p], gsem[p])

    def wait_gather(p):
        pltpu.make_async_copy(ps_hbm.at[isv[p]], a[p], gsem[p]).wait()
        pltpu.make_async_copy(pd_hbm.at[idv[p]], b[p], gsem[p]).wait()

    def issue_scatter(p):
        pltpu.async_copy(m[p], agg_sh.at[d2[p]], ssem[p], add=True)

    def wait_scatter(p):
        pltpu.make_async_copy(m[p], agg_sh.at[d2[p]], ssem[p]).wait()

    def save_idx(p):
        for k in range(CH // LANES):
            sl = pl.ds(k * LANES, LANES)
            d2[p][sl] = idv[p][sl]

    def compute(p):
        ap, bp, mp = a[p], b[p], m[p]

        def _crow(r, carry):
            for k in range(H // LANES):
                sl = pl.ds(k * LANES, LANES)
                mp[r, sl] = jnp.maximum(ap[r, sl] + bp[r, sl], 0.0)
            return carry
        lax.fori_loop(0, CH, _crow, 0)

    issue_idx(0, 0)
    issue_idx(1, 1)
    wait_idx(0)
    issue_gather(0)

    def _macro(t, carry):
        @pl.when(t >= 1)
        def _():
            wait_scatter(0)
        wait_idx(1)
        issue_gather(1)
        wait_gather(0)
        save_idx(0)

        @pl.when(t < NCHUNK // 2 - 1)
        def _():
            issue_idx(2 * t + 2, 0)
        compute(0)
        issue_scatter(0)

        @pl.when(t >= 1)
        def _():
            wait_scatter(1)

        @pl.when(t < NCHUNK // 2 - 1)
        def _():
            wait_idx(0)
            issue_gather(0)
        wait_gather(1)
        save_idx(1)

        @pl.when(t < NCHUNK // 2 - 1)
        def _():
            issue_idx(2 * t + 3, 1)
        compute(1)
        issue_scatter(1)
        return carry
    lax.fori_loop(0, NCHUNK // 2, _macro, 0)

    wait_scatter(0)
    wait_scatter(1)
    plsc.subcore_barrier()

    def _wtile(t, carry):
        off = (s + t * NS) * CH
        pltpu.sync_copy(agg_sh.at[pl.ds(off, CH)], m0)
        pltpu.sync_copy(m0, out_hbm.at[c, pl.ds(off, CH)])
        return carry
    lax.fori_loop(0, ntiles, _wtile, 0)


def _edge(ps, pd, src, dst):
    mesh = plsc.VectorSubcoreMesh(core_axis_name="c", subcore_axis_name="s")
    f = pl.kernel(
        _edge_body,
        out_type=jax.ShapeDtypeStruct((NC, NP, H), jnp.float32),
        mesh=mesh,
        scratch_types=[
            pltpu.VMEM((CH, H), jnp.float32),
            pltpu.VMEM((CH, H), jnp.float32),
            pltpu.VMEM((CH, H), jnp.float32),
            pltpu.VMEM((CH, H), jnp.float32),
            pltpu.VMEM((CH, H), jnp.float32),
            pltpu.VMEM((CH, H), jnp.float32),
            pltpu.VMEM((CH,), jnp.int32),
            pltpu.VMEM((CH,), jnp.int32),
            pltpu.VMEM((CH,), jnp.int32),
            pltpu.VMEM((CH,), jnp.int32),
            pltpu.VMEM((CH,), jnp.int32),
            pltpu.VMEM((CH,), jnp.int32),
            pltpu.VMEM((TAIL,), jnp.int32),
            pltpu.VMEM((TAIL,), jnp.int32),
            pltpu.VMEM_SHARED((NP, H), jnp.float32),
            pltpu.SemaphoreType.DMA,
            pltpu.SemaphoreType.DMA,
            pltpu.SemaphoreType.DMA,
            pltpu.SemaphoreType.DMA,
            pltpu.SemaphoreType.DMA,
            pltpu.SemaphoreType.DMA,
        ],
    )
    return f(ps, pd, src, dst)


def _final_body(s_ref, agg_ref, w_ref, o_ref):
    a = agg_ref[0] + agg_ref[1]
    o_ref[...] = jnp.maximum(
        s_ref[...] + jnp.dot(a, w_ref[...], preferred_element_type=jnp.float32),
        0.0)


def _final(s, agg2, w_agg):
    grid = N // RB
    return pl.pallas_call(
        _final_body,
        grid=(grid,),
        in_specs=[
            pl.BlockSpec((RB, H), lambda i: (i, 0)),
            pl.BlockSpec((NC, RB, H), lambda i: (0, i, 0)),
            pl.BlockSpec((H, H), lambda i: (0, 0)),
        ],
        out_specs=pl.BlockSpec((RB, H), lambda i: (i, 0)),
        out_shape=jax.ShapeDtypeStruct((N, H), jnp.float32),
    )(s, agg2, w_agg)


def kernel(x, edge_index, hidden, W_src, W_dst, W_self, W_agg):
    ei = edge_index.astype(jnp.int32)
    src, dst = ei[0], ei[1]
    wx = jnp.concatenate([W_src[:D], W_dst[:D], W_self[:D]], axis=1)
    wh = jnp.concatenate([W_src[D:], W_dst[D:], W_self[D:]], axis=1)
    ps, pd, s = _proj(x, hidden, wx, wh)
    agg2 = _edge(ps, pd, src, dst)
    return _final(s, agg2, W_agg)
